# baseline (speedup 1.0000x reference)
import functools

import jax
import jax.numpy as jnp
from jax import lax
from jax.experimental import pallas as pl
from jax.experimental.pallas import tpu as pltpu
from jax.experimental.pallas import tpu_sc as plsc

N = 100000
B = 64
C_IN = 256
C_MID = 256
C_OUT = 512

R = 2000
NB = N // R

NW = 32
CH = 3136
N_PAD = NW * CH
NV = CH // 16

_NEG_INF = float("-inf")


def _center_sc_body(p4t_hbm, ids_hbm, out_hbm, p4_v, ids_v, acc_v, red_v):
    wid = lax.axis_index("s") * 2 + lax.axis_index("c")
    base = wid * CH
    for c in range(4):
        pltpu.sync_copy(p4t_hbm.at[pl.ds(c * N_PAD + base, CH)],
                        p4_v.at[pl.ds(c * CH, CH)])
    pltpu.sync_copy(ids_hbm.at[pl.ds(base, CH)], ids_v)

    zeros16 = jnp.zeros((16,), jnp.float32)

    def zero_step(j, carry):
        acc_v[pl.ds(j * 16, 16)] = zeros16
        return carry

    lax.fori_loop(0, 256, zero_step, 0)

    lane = lax.iota(jnp.int32, 16) * 256

    def row_step(i, carry):
        idv = ids_v[pl.ds(i * 16, 16)]
        bank = lane + idv
        for c in range(4):
            idx = bank + c * 64
            cur = plsc.load_gather(acc_v, [idx])
            plsc.store_scatter(acc_v, [idx],
                               cur + p4_v[pl.ds(c * CH + i * 16, 16)])
        return carry

    lax.fori_loop(0, NV, row_step, 0)

    for j in range(16):
        tot = acc_v[pl.ds(j * 16, 16)]
        for k in range(1, 16):
            tot = tot + acc_v[pl.ds(k * 256 + j * 16, 16)]
        red_v[pl.ds(j * 16, 16)] = tot

    pltpu.sync_copy(red_v, out_hbm.at[pl.ds(wid * 4 * B, 4 * B)])


_center_sc = functools.partial(
    pl.kernel,
    out_type=jax.ShapeDtypeStruct((NW * 4 * B,), jnp.float32),
    mesh=plsc.VectorSubcoreMesh(core_axis_name="c", subcore_axis_name="s"),
    compiler_params=pltpu.CompilerParams(needs_layout_passes=False),
    scratch_types=[
        pltpu.VMEM((4 * CH,), jnp.float32),
        pltpu.VMEM((CH,), jnp.int32),
        pltpu.VMEM((16 * 256,), jnp.float32),
        pltpu.VMEM((4 * B,), jnp.float32),
    ],
)(_center_sc_body)


def _main_body(ids_ref, feat_ref, pos_ref, w1a_ref, w1b_ref, aggr_ref):
    i = pl.program_id(0)

    @pl.when(i == 0)
    def _():
        aggr_ref[...] = jnp.full((B, C_MID), _NEG_INF, jnp.float32)

    ids = ids_ref[0]
    a = jax.lax.dot_general(
        feat_ref[...].astype(jnp.bfloat16), w1a_ref[...],
        (((1,), (0,)), ((), ())), preferred_element_type=jnp.float32)
    a += jax.lax.dot_general(
        pos_ref[...].astype(jnp.bfloat16), w1b_ref[...],
        (((1,), (0,)), ((), ())), preferred_element_type=jnp.float32)

    s_lo = ids_ref[0, 0, 0]
    s_hi = ids_ref[0, R - 1, 0]

    def seg_step(s, carry):
        red = jnp.max(jnp.where(ids == s, a, _NEG_INF), axis=0, keepdims=True)
        aggr_ref[pl.ds(s, 1), :] = jnp.maximum(aggr_ref[pl.ds(s, 1), :], red)
        return carry

    jax.lax.fori_loop(s_lo, s_hi + 1, seg_step, 0)


def _epi_body(aggr_ref, sums_ref, w1b_ref, b1_ref, w2_ref, b2_ref, eps_ref,
              z_ref, mu_ref, sig_ref):
    sums = jnp.sum(sums_ref[...], axis=0)
    cnt = jnp.maximum(sums[3:4, :], 1.0)
    center_t = sums[0:3, :] / cnt
    cadj = b1_ref[...] - jax.lax.dot_general(
        center_t, w1b_ref[...], (((0,), (0,)), ((), ())),
        preferred_element_type=jnp.float32)
    agg = jnp.maximum(aggr_ref[...] + cadj, 0.0)
    out = jax.lax.dot_general(
        agg, w2_ref[...], (((1,), (0,)), ((), ())),
        preferred_element_type=jnp.float32) + b2_ref[...]
    mu = out[:, :C_MID]
    sr = out[:, C_MID:]
    sigma = jnp.maximum(sr, 0.0) + jnp.log1p(jnp.exp(-jnp.abs(sr)))
    mu_ref[...] = mu
    sig_ref[...] = sigma
    z_ref[...] = mu + sigma * eps_ref[...]


@jax.jit
def _run(pos, feature, ids_col, p4t, idsp, W1a, W1b, b1r, W2, b2r, eps):
    sums32 = _center_sc(p4t, idsp).reshape(NW, 4, B)

    agg_raw = pl.pallas_call(
        _main_body,
        grid=(NB,),
        in_specs=[
            pl.BlockSpec((1, R, 1), lambda i: (i, 0, 0)),
            pl.BlockSpec((R, C_IN), lambda i: (i, 0)),
            pl.BlockSpec((R, 3), lambda i: (i, 0)),
            pl.BlockSpec((C_IN, C_MID), lambda i: (0, 0)),
            pl.BlockSpec((3, C_MID), lambda i: (0, 0)),
        ],
        out_specs=pl.BlockSpec((B, C_MID), lambda i: (0, 0)),
        out_shape=jax.ShapeDtypeStruct((B, C_MID), jnp.float32),
    )(ids_col, feature, pos,
      W1a.astype(jnp.bfloat16), W1b.astype(jnp.bfloat16))

    z, mu, sigma = pl.pallas_call(
        _epi_body,
        in_specs=[
            pl.BlockSpec((B, C_MID), lambda: (0, 0)),
            pl.BlockSpec((NW, 4, B), lambda: (0, 0, 0)),
            pl.BlockSpec((3, C_MID), lambda: (0, 0)),
            pl.BlockSpec((1, C_MID), lambda: (0, 0)),
            pl.BlockSpec((C_MID, C_OUT), lambda: (0, 0)),
            pl.BlockSpec((1, C_OUT), lambda: (0, 0)),
            pl.BlockSpec((B, C_MID), lambda: (0, 0)),
        ],
        out_specs=[
            pl.BlockSpec((B, C_MID), lambda: (0, 0)),
            pl.BlockSpec((B, C_MID), lambda: (0, 0)),
            pl.BlockSpec((B, C_MID), lambda: (0, 0)),
        ],
        out_shape=[
            jax.ShapeDtypeStruct((B, C_MID), jnp.float32),
            jax.ShapeDtypeStruct((B, C_MID), jnp.float32),
            jax.ShapeDtypeStruct((B, C_MID), jnp.float32),
        ],
    )(agg_raw, sums32, W1b, b1r, W2, b2r, eps)
    return z, mu, sigma


def kernel(pos, feature, batch, W1, b1, W2, b2):
    ids = batch.astype(jnp.int32)
    ids_col = ids.reshape(NB, R, 1)
    p4t = jnp.concatenate([pos.T, jnp.ones((1, N), jnp.float32)], axis=0)
    p4t = jnp.pad(p4t, ((0, 0), (0, N_PAD - N))).reshape(4 * N_PAD)
    idsp = jnp.pad(ids, (0, N_PAD - N), constant_values=B - 1)
    W1a = W1[:C_IN]
    W1b = W1[C_IN:]
    b1r = b1.reshape(1, C_MID)
    b2r = b2.reshape(1, C_OUT)
    eps = jax.random.normal(jax.random.key(1), (B, C_MID), dtype=jnp.float32)
    z, mu, sigma = _run(pos, feature, ids_col, p4t, idsp, W1a, W1b, b1r, W2,
                        b2r, eps)
    pos_center_batch = jnp.arange(B, dtype=jnp.int64)
    return (z, mu, sigma, pos_center_batch)
